# Initial kernel scaffold; baseline (speedup 1.0000x reference)
import jax
import jax.numpy as jnp
from jax.experimental import pallas as pl


def kernel(x, x_len, codebooks):
    raise NotImplementedError("write your pallas kernel here")


# speedup vs baseline: 26.2455x; 26.2455x over previous
import functools

import jax
import jax.numpy as jnp
from jax import lax
from jax.experimental import pallas as pl
from jax.experimental.pallas import tpu as pltpu
from jax.experimental.pallas import tpu_sc as plsc

B, Q, T, K, D = 16, 8, 2048, 1024, 128
L = 16
NW = 32
C = 128
NB = 6
POS_PER_W = (B * T) // NW
CHUNKS_PER_W = POS_PER_W // C
SEGS = D // L


def kernel(x, x_len, codebooks):
    cb_flat = codebooks.reshape(Q * K, D)
    mesh = plsc.VectorSubcoreMesh(core_axis_name="c", subcore_axis_name="s")

    @functools.partial(
        pl.kernel,
        out_type=jax.ShapeDtypeStruct((B, T, D), jnp.float32),
        mesh=mesh,
        compiler_params=pltpu.CompilerParams(needs_layout_passes=False),
        scratch_types=[
            pltpu.VMEM((Q, C), jnp.int32),
            [pltpu.VMEM((C, D), jnp.float32) for _ in range(NB)],
            pltpu.VMEM((B,), jnp.int32),
            pltpu.SemaphoreType.DMA,
        ],
    )
    def run(x_hbm, xlen_hbm, cb_hbm, out_hbm, idx_v, bufs, xlen_v, sem):
        wid = lax.axis_index("s") * 2 + lax.axis_index("c")
        b = wid // 2
        tbase = (wid % 2) * POS_PER_W

        pltpu.sync_copy(xlen_hbm, xlen_v)
        xlv = plsc.load_gather(xlen_v, [jnp.full((L,), b, jnp.int32)])

        def accum(dsts, t0=None):
            def pos_body(p, _):
                mf = None
                if t0 is not None:
                    tvec = lax.broadcast_in_dim(t0 + p, (L,), ())
                    mf = jnp.where(tvec < xlv, jnp.float32(1.0), jnp.float32(0.0))
                for s in range(SEGS):
                    sl = pl.ds(s * L, L)
                    v = bufs[0][p, sl]
                    for j in dsts:
                        v = v + bufs[j][p, sl]
                    if mf is not None:
                        v = v * mf
                    bufs[0][p, sl] = v
                return _
            lax.fori_loop(0, C, pos_body, None)

        def chunk_body(k, _):
            t0 = tbase + k * C
            pltpu.sync_copy(x_hbm.at[b, :, pl.ds(t0, C)], idx_v)
            for q in range(1, Q):
                for s in range(C // L):
                    sl = pl.ds(s * L, L)
                    idx_v[q, sl] = idx_v[q, sl] + q * K
            copies = [
                pltpu.async_copy(cb_hbm.at[idx_v.at[q]], bufs[q], sem)
                for q in range(NB)
            ]
            for cp in copies:
                cp.wait()
            accum(range(1, NB))
            copies = [
                pltpu.async_copy(cb_hbm.at[idx_v.at[NB + j]], bufs[1 + j], sem)
                for j in range(Q - NB)
            ]
            for cp in copies:
                cp.wait()
            accum(range(1, 1 + Q - NB), t0)
            pltpu.sync_copy(bufs[0], out_hbm.at[b, pl.ds(t0, C)])
            return _

        lax.fori_loop(0, CHUNKS_PER_W, chunk_body, None)

    return run(x, x_len, cb_flat)


# speedup vs baseline: 34.9840x; 1.3330x over previous
import functools

import jax
import jax.numpy as jnp
from jax import lax
from jax.experimental import pallas as pl
from jax.experimental.pallas import tpu as pltpu
from jax.experimental.pallas import tpu_sc as plsc

B, Q, T, K, D = 16, 8, 2048, 1024, 128
L = 16
NW = 32
C = 128
NB = 6
POS_PER_W = (B * T) // NW
CHUNKS_PER_W = POS_PER_W // C
SEGS = D // L


def kernel(x, x_len, codebooks):
    cb_flat = codebooks.reshape(Q * K, D)
    mesh = plsc.VectorSubcoreMesh(core_axis_name="c", subcore_axis_name="s")

    @functools.partial(
        pl.kernel,
        out_type=jax.ShapeDtypeStruct((B, T, D), jnp.float32),
        mesh=mesh,
        compiler_params=pltpu.CompilerParams(needs_layout_passes=False),
        scratch_types=[
            pltpu.VMEM((Q, C), jnp.int32),
            [pltpu.VMEM((C, D), jnp.float32) for _ in range(NB)],
            pltpu.VMEM((B,), jnp.int32),
            pltpu.SemaphoreType.DMA,
        ],
    )
    def run(x_hbm, xlen_hbm, cb_hbm, out_hbm, idx_v, bufs, xlen_v, sem):
        wid = lax.axis_index("s") * 2 + lax.axis_index("c")
        b = wid // 2
        tbase = (wid % 2) * POS_PER_W

        pltpu.sync_copy(xlen_hbm, xlen_v)
        xlv = plsc.load_gather(xlen_v, [jnp.full((L,), b, jnp.int32)])

        zeros = jnp.zeros((L,), jnp.float32)

        def chunk_body(k, _):
            t0 = tbase + k * C
            pltpu.sync_copy(x_hbm.at[b, :, pl.ds(t0, C)], idx_v)
            for q in range(1, Q):
                for s in range(C // L):
                    sl = pl.ds(s * L, L)
                    idx_v[q, sl] = idx_v[q, sl] + q * K

            def zero_body(p, _):
                for s in range(SEGS):
                    bufs[0][p, pl.ds(s * L, L)] = zeros
                return _
            lax.fori_loop(0, C, zero_body, None)

            copies = [
                pltpu.async_copy(cb_hbm.at[idx_v.at[q]], bufs[0], sem, add=True)
                for q in range(Q)
            ]
            for cp in copies:
                cp.wait()

            def mask_body(p, _):
                tvec = lax.broadcast_in_dim(t0 + p, (L,), ())
                mf = jnp.where(tvec < xlv, jnp.float32(1.0), jnp.float32(0.0))
                for s in range(SEGS):
                    sl = pl.ds(s * L, L)
                    bufs[0][p, sl] = bufs[0][p, sl] * mf
                return _
            lax.fori_loop(0, C, mask_body, None)
            pltpu.sync_copy(bufs[0], out_hbm.at[b, pl.ds(t0, C)])
            return _

        lax.fori_loop(0, CHUNKS_PER_W, chunk_body, None)

    return run(x, x_len, cb_flat)


# speedup vs baseline: 40.7102x; 1.1637x over previous
import functools

import jax
import jax.numpy as jnp
from jax import lax
from jax.experimental import pallas as pl
from jax.experimental.pallas import tpu as pltpu
from jax.experimental.pallas import tpu_sc as plsc

B, Q, T, K, D = 16, 8, 2048, 1024, 128
L = 16
NW = 32
C = 128
NACC = 3
NIDX = 2
POS_PER_W = (B * T) // NW
NCHUNK = POS_PER_W // C
SEGS = D // L


def kernel(x, x_len, codebooks):
    cb_flat = codebooks.reshape(Q * K, D)
    mesh = plsc.VectorSubcoreMesh(core_axis_name="c", subcore_axis_name="s")

    @functools.partial(
        pl.kernel,
        out_type=jax.ShapeDtypeStruct((B, T, D), jnp.float32),
        mesh=mesh,
        compiler_params=pltpu.CompilerParams(needs_layout_passes=False),
        scratch_types=[
            [pltpu.VMEM((Q, C), jnp.int32) for _ in range(NIDX)],
            [pltpu.VMEM((C, D), jnp.float32) for _ in range(NACC)],
            pltpu.VMEM((B,), jnp.int32),
            [pltpu.SemaphoreType.DMA for _ in range(2)],
            [pltpu.SemaphoreType.DMA for _ in range(NACC)],
        ],
    )
    def run(x_hbm, xlen_hbm, cb_hbm, out_hbm, idxs, accs, xlen_v,
            gsems, osems):
        wid = lax.axis_index("s") * 2 + lax.axis_index("c")
        b = wid // 2
        tbase = (wid % 2) * POS_PER_W

        pltpu.sync_copy(xlen_hbm, xlen_v)
        xlv = plsc.load_gather(xlen_v, [jnp.full((L,), b, jnp.int32)])

        zeros = jnp.zeros((L,), jnp.float32)

        def zero_acc(acc):
            def zbody(p, _):
                for s in range(SEGS):
                    acc[p, pl.ds(s * L, L)] = zeros
                return _
            lax.fori_loop(0, C, zbody, None)

        t0s = [tbase + k * C for k in range(NCHUNK)]
        gathers = [None] * NCHUNK
        outs = [None] * NCHUNK

        def stage(k):
            idx_v = idxs[k % NIDX]
            acc = accs[k % NACC]
            pltpu.sync_copy(x_hbm.at[b, :, pl.ds(t0s[k], C)], idx_v)
            for q in range(1, Q):
                for s in range(C // L):
                    sl = pl.ds(s * L, L)
                    idx_v[q, sl] = idx_v[q, sl] + q * K
            if k >= NACC:
                outs[k - NACC].wait()
            zero_acc(acc)
            gathers[k] = [
                pltpu.async_copy(cb_hbm.at[idx_v.at[q]], acc, gsems[k % 2],
                                 add=True)
                for q in range(Q)
            ]

        def finish(k):
            acc = accs[k % NACC]
            for cp in gathers[k]:
                cp.wait()
            t0 = t0s[k]

            def mask_body(p, _):
                tvec = lax.broadcast_in_dim(t0 + p, (L,), ())
                mf = jnp.where(tvec < xlv, jnp.float32(1.0), jnp.float32(0.0))
                for s in range(SEGS):
                    sl = pl.ds(s * L, L)
                    acc[p, sl] = acc[p, sl] * mf
                return _
            lax.fori_loop(0, C, mask_body, None)
            outs[k] = pltpu.async_copy(acc, out_hbm.at[b, pl.ds(t0, C)],
                                       osems[k % NACC])

        stage(0)
        stage(1)
        for k in range(NCHUNK):
            finish(k)
            if k + 2 < NCHUNK:
                stage(k + 2)
        for k in range(NCHUNK - NACC, NCHUNK):
            outs[k].wait()

    return run(x, x_len, cb_flat)
